# baseline (speedup 1.0000x reference)
import functools

import jax
import jax.numpy as jnp
from jax.experimental import pallas as pl
from jax.experimental.pallas import tpu as pltpu


def _phrase_kernel(s1_ref, s2_ref, wenc_ref, benc_ref, w1_ref, b1_ref,
                   w2_ref, b2_ref, out_ref, acc1_ref, acc2_ref,
                   *, odim, nl):
    l = pl.program_id(1)

    p1 = jnp.sum(s1_ref[...], axis=0)
    p2 = jnp.sum(s2_ref[...], axis=0)

    @pl.when(l == 0)
    def _init():
        acc1_ref[...] = p1
        acc2_ref[...] = p2

    @pl.when(l != 0)
    def _accum():
        acc1_ref[...] += p1
        acc2_ref[...] += p2

    @pl.when(l == nl - 1)
    def _epilogue():
        wenc = wenc_ref[...]
        benc = benc_ref[...]
        h1 = jnp.tanh(jnp.dot(acc1_ref[...], wenc,
                              preferred_element_type=jnp.float32) + benc)
        h2 = jnp.tanh(jnp.dot(acc2_ref[...], wenc,
                              preferred_element_type=jnp.float32) + benc)

        w1 = w1_ref[...]
        z = (jnp.dot(h1, w1[0 * odim:1 * odim, :],
                     preferred_element_type=jnp.float32)
             + jnp.dot(h2, w1[1 * odim:2 * odim, :],
                       preferred_element_type=jnp.float32)
             + jnp.dot(jnp.abs(h1 - h2), w1[2 * odim:3 * odim, :],
                       preferred_element_type=jnp.float32)
             + jnp.dot(h1 * h2, w1[3 * odim:4 * odim, :],
                       preferred_element_type=jnp.float32)
             + b1_ref[...])
        z = jnp.maximum(z, 0.0)

        logits = jnp.sum(z * w2_ref[...], axis=-1) + b2_ref[0]
        out_ref[...] = (1.0 / (1.0 + jnp.exp(-logits)))[None, :]


def kernel(seq1, seq2, wenc, benc, w1, b1, w2, b2):
    L, B, idim = seq1.shape
    odim = wenc.shape[1]

    bt = B if B <= 128 else 128
    assert B % bt == 0
    nb = B // bt

    lc = L
    for cand in (8, 4, 2, 1):
        if L % cand == 0:
            lc = cand
            break
    nl = L // lc

    wenc_scaled = wenc * (1.0 / L)
    w2_row = w2.reshape(1, odim)
    b2_s = b2.reshape(1)

    const = lambda shape: pl.BlockSpec(shape, lambda b, t: (0, 0))

    out = pl.pallas_call(
        functools.partial(_phrase_kernel, odim=odim, nl=nl),
        out_shape=jax.ShapeDtypeStruct((1, B), jnp.float32),
        grid=(nb, nl),
        in_specs=[
            pl.BlockSpec((lc, bt, idim), lambda b, t: (t, b, 0)),
            pl.BlockSpec((lc, bt, idim), lambda b, t: (t, b, 0)),
            const((idim, odim)),
            const((1, odim)),
            const((4 * odim, odim)),
            const((1, odim)),
            const((1, odim)),
            pl.BlockSpec(memory_space=pltpu.MemorySpace.SMEM),
        ],
        out_specs=pl.BlockSpec((1, bt), lambda b, t: (0, b)),
        scratch_shapes=[
            pltpu.VMEM((bt, idim), jnp.float32),
            pltpu.VMEM((bt, idim), jnp.float32),
        ],
        compiler_params=pltpu.CompilerParams(
            dimension_semantics=("parallel", "arbitrary"),
            vmem_limit_bytes=32 << 20),
    )(seq1, seq2, wenc_scaled, benc, w1, b1, w2_row, b2_s)

    return out.reshape(B, 1)


# speedup vs baseline: 1.3341x; 1.3341x over previous
import functools

import jax
import jax.numpy as jnp
from jax.experimental import pallas as pl
from jax.experimental.pallas import tpu as pltpu


def _phrase_kernel(s1_ref, s2_ref, wenc_ref, benc_ref, w1_ref, b1_ref,
                   w2_ref, b2_ref, out_ref, acc1_ref, acc2_ref,
                   *, odim, nl):
    l = pl.program_id(1)

    p1 = jnp.sum(s1_ref[...], axis=0)
    p2 = jnp.sum(s2_ref[...], axis=0)

    @pl.when(l == 0)
    def _init():
        acc1_ref[...] = p1
        acc2_ref[...] = p2

    @pl.when(l != 0)
    def _accum():
        acc1_ref[...] += p1
        acc2_ref[...] += p2

    @pl.when(l == nl - 1)
    def _epilogue():
        wenc = wenc_ref[...]
        benc = benc_ref[...]
        h1 = jnp.tanh(jnp.dot(acc1_ref[...], wenc,
                              preferred_element_type=jnp.float32) + benc)
        h2 = jnp.tanh(jnp.dot(acc2_ref[...], wenc,
                              preferred_element_type=jnp.float32) + benc)

        w1 = w1_ref[...]
        z = (jnp.dot(h1, w1[0 * odim:1 * odim, :],
                     preferred_element_type=jnp.float32)
             + jnp.dot(h2, w1[1 * odim:2 * odim, :],
                       preferred_element_type=jnp.float32)
             + jnp.dot(jnp.abs(h1 - h2), w1[2 * odim:3 * odim, :],
                       preferred_element_type=jnp.float32)
             + jnp.dot(h1 * h2, w1[3 * odim:4 * odim, :],
                       preferred_element_type=jnp.float32)
             + b1_ref[...])
        z = jnp.maximum(z, 0.0)

        logits = jnp.sum(z * w2_ref[...], axis=-1) + b2_ref[0]
        out_ref[...] = (1.0 / (1.0 + jnp.exp(-logits)))[None, :]


def kernel(seq1, seq2, wenc, benc, w1, b1, w2, b2):
    L, B, idim = seq1.shape
    odim = wenc.shape[1]

    bt = B if B <= 512 else 512
    assert B % bt == 0
    nb = B // bt

    lc = L
    for cand in (4, 2, 1):
        if L % cand == 0:
            lc = cand
            break
    nl = L // lc

    wenc_scaled = wenc * (1.0 / L)
    w2_row = w2.reshape(1, odim)
    b2_s = b2.reshape(1)

    const = lambda shape: pl.BlockSpec(shape, lambda b, t: (0, 0))

    out = pl.pallas_call(
        functools.partial(_phrase_kernel, odim=odim, nl=nl),
        out_shape=jax.ShapeDtypeStruct((1, B), jnp.float32),
        grid=(nb, nl),
        in_specs=[
            pl.BlockSpec((lc, bt, idim), lambda b, t: (t, b, 0)),
            pl.BlockSpec((lc, bt, idim), lambda b, t: (t, b, 0)),
            const((idim, odim)),
            const((1, odim)),
            const((4 * odim, odim)),
            const((1, odim)),
            const((1, odim)),
            pl.BlockSpec(memory_space=pltpu.MemorySpace.SMEM),
        ],
        out_specs=pl.BlockSpec((1, bt), lambda b, t: (0, b)),
        scratch_shapes=[
            pltpu.VMEM((bt, idim), jnp.float32),
            pltpu.VMEM((bt, idim), jnp.float32),
        ],
        compiler_params=pltpu.CompilerParams(
            dimension_semantics=("parallel", "arbitrary"),
            vmem_limit_bytes=32 << 20),
    )(seq1, seq2, wenc_scaled, benc, w1, b1, w2_row, b2_s)

    return out.reshape(B, 1)


# speedup vs baseline: 1.5646x; 1.1728x over previous
import functools

import jax
import jax.numpy as jnp
from jax.experimental import pallas as pl
from jax.experimental.pallas import tpu as pltpu


def _phrase_kernel(s1_ref, s2_ref, wenc_ref, benc_ref, w1_ref, b1_ref,
                   w2_ref, b2_ref, out_ref, acc1_ref, acc2_ref,
                   *, odim, nl):
    l = pl.program_id(1)

    p1 = jnp.sum(s1_ref[...], axis=0)
    p2 = jnp.sum(s2_ref[...], axis=0)

    @pl.when(l == 0)
    def _init():
        acc1_ref[...] = p1
        acc2_ref[...] = p2

    @pl.when(l != 0)
    def _accum():
        acc1_ref[...] += p1
        acc2_ref[...] += p2

    @pl.when(l == nl - 1)
    def _epilogue():
        wenc = wenc_ref[...]
        benc = benc_ref[...]
        h1 = jnp.tanh(jnp.dot(acc1_ref[...], wenc,
                              preferred_element_type=jnp.float32) + benc)
        h2 = jnp.tanh(jnp.dot(acc2_ref[...], wenc,
                              preferred_element_type=jnp.float32) + benc)

        w1 = w1_ref[...]
        z = (jnp.dot(h1, w1[0 * odim:1 * odim, :],
                     preferred_element_type=jnp.float32)
             + jnp.dot(h2, w1[1 * odim:2 * odim, :],
                       preferred_element_type=jnp.float32)
             + jnp.dot(jnp.abs(h1 - h2), w1[2 * odim:3 * odim, :],
                       preferred_element_type=jnp.float32)
             + jnp.dot(h1 * h2, w1[3 * odim:4 * odim, :],
                       preferred_element_type=jnp.float32)
             + b1_ref[...])
        z = jnp.maximum(z, 0.0)

        logits = jnp.sum(z * w2_ref[...], axis=-1) + b2_ref[0]
        out_ref[...] = (1.0 / (1.0 + jnp.exp(-logits)))[None, :]


def kernel(seq1, seq2, wenc, benc, w1, b1, w2, b2):
    L, B, idim = seq1.shape
    odim = wenc.shape[1]

    bt = B if B <= 512 else 512
    assert B % bt == 0
    nb = B // bt

    lc = L
    for cand in (8, 4, 2, 1):
        if L % cand == 0:
            lc = cand
            break
    nl = L // lc

    wenc_scaled = wenc * (1.0 / L)
    w2_row = w2.reshape(1, odim)
    b2_s = b2.reshape(1)

    const = lambda shape: pl.BlockSpec(shape, lambda b, t: (0, 0))

    out = pl.pallas_call(
        functools.partial(_phrase_kernel, odim=odim, nl=nl),
        out_shape=jax.ShapeDtypeStruct((1, B), jnp.float32),
        grid=(nb, nl),
        in_specs=[
            pl.BlockSpec((lc, bt, idim), lambda b, t: (t, b, 0)),
            pl.BlockSpec((lc, bt, idim), lambda b, t: (t, b, 0)),
            const((idim, odim)),
            const((1, odim)),
            const((4 * odim, odim)),
            const((1, odim)),
            const((1, odim)),
            pl.BlockSpec(memory_space=pltpu.MemorySpace.SMEM),
        ],
        out_specs=pl.BlockSpec((1, bt), lambda b, t: (0, b)),
        scratch_shapes=[
            pltpu.VMEM((bt, idim), jnp.float32),
            pltpu.VMEM((bt, idim), jnp.float32),
        ],
        compiler_params=pltpu.CompilerParams(
            dimension_semantics=("parallel", "arbitrary"),
            vmem_limit_bytes=32 << 20),
    )(seq1, seq2, wenc_scaled, benc, w1, b1, w2_row, b2_s)

    return out.reshape(B, 1)


# speedup vs baseline: 2.2769x; 1.4553x over previous
import functools

import jax
import jax.numpy as jnp
from jax.experimental import pallas as pl
from jax.experimental.pallas import tpu as pltpu


def _phrase_kernel(s1_ref, s2_ref, wenc_ref, benc_ref, w1_ref, b1_ref,
                   w2_ref, b2_ref, out_ref, acc1_ref, acc2_ref,
                   *, odim, nl):
    l = pl.program_id(1)

    p1 = jnp.sum(s1_ref[...], axis=0)
    p2 = jnp.sum(s2_ref[...], axis=0)

    @pl.when(l == 0)
    def _init():
        acc1_ref[...] = p1
        acc2_ref[...] = p2

    @pl.when(l != 0)
    def _accum():
        acc1_ref[...] += p1
        acc2_ref[...] += p2

    @pl.when(l == nl - 1)
    def _epilogue():
        wenc = wenc_ref[...]
        benc = benc_ref[...]
        h1 = jnp.tanh(jnp.dot(acc1_ref[...], wenc,
                              preferred_element_type=jnp.float32) + benc)
        h2 = jnp.tanh(jnp.dot(acc2_ref[...], wenc,
                              preferred_element_type=jnp.float32) + benc)

        w1 = w1_ref[...]
        z = (jnp.dot(h1, w1[0 * odim:1 * odim, :],
                     preferred_element_type=jnp.float32)
             + jnp.dot(h2, w1[1 * odim:2 * odim, :],
                       preferred_element_type=jnp.float32)
             + jnp.dot(jnp.abs(h1 - h2), w1[2 * odim:3 * odim, :],
                       preferred_element_type=jnp.float32)
             + jnp.dot(h1 * h2, w1[3 * odim:4 * odim, :],
                       preferred_element_type=jnp.float32)
             + b1_ref[...])
        z = jnp.maximum(z, 0.0)

        logits = jnp.sum(z * w2_ref[...], axis=-1) + b2_ref[0]
        out_ref[...] = (1.0 / (1.0 + jnp.exp(-logits)))[None, :]


def kernel(seq1, seq2, wenc, benc, w1, b1, w2, b2):
    L, B, idim = seq1.shape
    odim = wenc.shape[1]

    bt = B if B <= 512 else 512
    assert B % bt == 0
    nb = B // bt

    lc = L
    nl = L // lc

    wenc_scaled = wenc * (1.0 / L)
    w2_row = w2.reshape(1, odim)
    b2_s = b2.reshape(1)

    const = lambda shape: pl.BlockSpec(shape, lambda b, t: (0, 0))

    out = pl.pallas_call(
        functools.partial(_phrase_kernel, odim=odim, nl=nl),
        out_shape=jax.ShapeDtypeStruct((1, B), jnp.float32),
        grid=(nb, nl),
        in_specs=[
            pl.BlockSpec((lc, bt, idim), lambda b, t: (t, b, 0)),
            pl.BlockSpec((lc, bt, idim), lambda b, t: (t, b, 0)),
            const((idim, odim)),
            const((1, odim)),
            const((4 * odim, odim)),
            const((1, odim)),
            const((1, odim)),
            pl.BlockSpec(memory_space=pltpu.MemorySpace.SMEM),
        ],
        out_specs=pl.BlockSpec((1, bt), lambda b, t: (0, b)),
        scratch_shapes=[
            pltpu.VMEM((bt, idim), jnp.float32),
            pltpu.VMEM((bt, idim), jnp.float32),
        ],
        compiler_params=pltpu.CompilerParams(
            dimension_semantics=("parallel", "arbitrary"),
            vmem_limit_bytes=56 << 20),
    )(seq1, seq2, wenc_scaled, benc, w1, b1, w2_row, b2_s)

    return out.reshape(B, 1)
